# Initial kernel scaffold; baseline (speedup 1.0000x reference)
import jax
import jax.numpy as jnp
from jax.experimental import pallas as pl


def kernel(x, edge_index, W1, b1, W2, b2, W3, b3):
    raise NotImplementedError("write your pallas kernel here")


# speedup vs baseline: 5.3397x; 5.3397x over previous
import functools

import jax
import jax.numpy as jnp
from jax import lax
from jax.experimental import pallas as pl
from jax.experimental.pallas import tpu as pltpu
from jax.experimental.pallas import tpu_sc as plsc

N = 10000
E = 160000
NT = 16
EPT = E // NT
EB = 80
RPT = 624
ZB = 208
TAIL = N - NT * RPT
BR = 1000


_SC_MESH = plsc.VectorSubcoreMesh(core_axis_name="c", subcore_axis_name="s")


def _zero_shared(s, zb_v, shared):
    for j in range(RPT // ZB):
        pltpu.sync_copy(zb_v, shared.at[pl.ds(s * RPT + j * ZB, ZB)])

    @pl.when(s == 0)
    def _():
        pltpu.sync_copy(zb_v.at[pl.ds(0, TAIL)], shared.at[pl.ds(NT * RPT, TAIL)])


def _write_out(s, shared, out_hbm, base):
    pltpu.sync_copy(shared.at[pl.ds(s * RPT, RPT)],
                    out_hbm.at[pl.ds(base + s * RPT, RPT)])

    @pl.when(s == 0)
    def _():
        pltpu.sync_copy(shared.at[pl.ds(NT * RPT, TAIL)],
                        out_hbm.at[pl.ds(base + NT * RPT, TAIL)])


@functools.partial(
    pl.kernel,
    out_type=jax.ShapeDtypeStruct((N, 128), jnp.float32),
    mesh=_SC_MESH,
    scratch_types=[
        pltpu.VMEM((EB,), jnp.int32),
        pltpu.VMEM((EB, 128), jnp.float32),
        pltpu.VMEM((ZB, 128), jnp.float32),
        pltpu.VMEM_SHARED((N, 128), jnp.float32),
    ],
)
def _deg_kernel(dst_hbm, deg_hbm, dst_v, ones_v, zb_v, shared):
    c = lax.axis_index("c")
    s = lax.axis_index("s")
    one16 = jnp.full((16,), 1.0, jnp.float32)
    zero16 = jnp.zeros((16,), jnp.float32)

    def fill(i, carry):
        for j in range(8):
            ones_v[i, pl.ds(j * 16, 16)] = one16
        return carry

    lax.fori_loop(0, EB, fill, 0)

    def fillz(i, carry):
        for j in range(8):
            zb_v[i, pl.ds(j * 16, 16)] = zero16
        return carry

    lax.fori_loop(0, ZB, fillz, 0)
    _zero_shared(s, zb_v, shared)
    plsc.subcore_barrier()

    def chunk(i, carry):
        base = s * EPT + i * EB
        pltpu.sync_copy(dst_hbm.at[pl.ds(base, EB)], dst_v)
        pltpu.sync_copy(ones_v, shared.at[dst_v], add=True)
        return carry

    lax.fori_loop(0, EPT // EB, chunk, 0)
    plsc.subcore_barrier()

    @pl.when(c == 0)
    def _():
        _write_out(s, shared, deg_hbm, 0)


def _make_scatter(G):
    gpc = G // 2

    @functools.partial(
        pl.kernel,
        out_type=jax.ShapeDtypeStruct((G * N, 128), jnp.float32),
        mesh=_SC_MESH,
        scratch_types=[
            pltpu.VMEM((EB,), jnp.int32),
            pltpu.VMEM((EB,), jnp.int32),
            pltpu.VMEM((EB, 128), jnp.float32),
            pltpu.VMEM((ZB, 128), jnp.float32),
            pltpu.VMEM_SHARED((N, 128), jnp.float32),
            pltpu.SemaphoreType.DMA,
        ],
    )
    def scatter(src_hbm, dst_hbm, y_hbm, z_hbm,
                dst_v, gidx_v, rows_v, zb_v, shared, sem):
        c = lax.axis_index("c")
        s = lax.axis_index("s")
        zero16 = jnp.zeros((16,), jnp.float32)

        def fillz(i, carry):
            for j in range(8):
                zb_v[i, pl.ds(j * 16, 16)] = zero16
            return carry

        lax.fori_loop(0, ZB, fillz, 0)

        for gp in range(gpc):
            g = c * gpc + gp
            _zero_shared(s, zb_v, shared)
            plsc.subcore_barrier()

            def chunk(i, carry):
                base = s * EPT + i * EB
                pltpu.sync_copy(dst_hbm.at[pl.ds(base, EB)], dst_v)
                pltpu.sync_copy(src_hbm.at[pl.ds(base, EB)], gidx_v)
                off = g * N

                def addoff(k, cc):
                    v = gidx_v[pl.ds(k * 16, 16)]
                    gidx_v[pl.ds(k * 16, 16)] = v + off
                    return cc

                lax.fori_loop(0, EB // 16, addoff, 0)
                pltpu.async_copy(y_hbm.at[gidx_v], rows_v, sem).wait()
                pltpu.sync_copy(rows_v, shared.at[dst_v], add=True)
                return carry

            lax.fori_loop(0, EPT // EB, chunk, 0)
            plsc.subcore_barrier()
            _write_out(s, shared, z_hbm, g * N)
            plsc.subcore_barrier()

    return scatter


_scatter4 = _make_scatter(4)
_scatter2 = _make_scatter(2)


def _dis(deg_ref):
    return lax.rsqrt(deg_ref[:, :1] + 1.0)


def _mm1_body(x_ref, w_ref, deg_ref, y_ref):
    xw = jnp.dot(x_ref[...], w_ref[...], preferred_element_type=jnp.float32)
    y_ref[...] = xw * _dis(deg_ref)


def _mm1(x, W1, deg16):
    return pl.pallas_call(
        _mm1_body,
        grid=(N // BR, 4),
        in_specs=[
            pl.BlockSpec((BR, 256), lambda i, g: (i, 0)),
            pl.BlockSpec((256, 128), lambda i, g: (0, g)),
            pl.BlockSpec((BR, 128), lambda i, g: (i, 0)),
        ],
        out_specs=pl.BlockSpec((BR, 128), lambda i, g: (g * (N // BR) + i, 0)),
        out_shape=jax.ShapeDtypeStruct((4 * N, 128), jnp.float32),
    )(x, W1, deg16)


def _mm23_body(kgrid, z_ref, y_ref, deg_ref, b_ref, w_ref, out_ref):
    k = pl.program_id(2)
    dis = _dis(deg_ref)
    h = jnp.maximum((z_ref[...] + y_ref[...]) * dis + b_ref[0], 0.0)
    part = jnp.dot(h, w_ref[...], preferred_element_type=jnp.float32)

    @pl.when(k == 0)
    def _():
        out_ref[...] = part

    @pl.when(k > 0)
    def _():
        out_ref[...] += part

    @pl.when(k == kgrid - 1)
    def _():
        out_ref[...] *= dis


def _mm23(z, y, deg16, b4, W, gout):
    r = N // BR
    return pl.pallas_call(
        functools.partial(_mm23_body, 4),
        grid=(r, gout, 4),
        in_specs=[
            pl.BlockSpec((BR, 128), lambda i, go, k: (k * r + i, 0)),
            pl.BlockSpec((BR, 128), lambda i, go, k: (k * r + i, 0)),
            pl.BlockSpec((BR, 128), lambda i, go, k: (i, 0)),
            pl.BlockSpec((1, 1, 128), lambda i, go, k: (k, 0, 0)),
            pl.BlockSpec((128, 128), lambda i, go, k: (k, go)),
        ],
        out_specs=pl.BlockSpec((BR, 128), lambda i, go, k: (go * r + i, 0)),
        out_shape=jax.ShapeDtypeStruct((gout * N, 128), jnp.float32),
    )(z, y, deg16, b4, W)


def _final_body(z_ref, y_ref, deg_ref, b_ref, out_ref):
    out_ref[...] = jnp.maximum(
        (z_ref[...] + y_ref[...]) * _dis(deg_ref) + b_ref[0], 0.0)


def _final(z, y, deg16, b2g):
    r = N // BR
    return pl.pallas_call(
        _final_body,
        grid=(r, 2),
        in_specs=[
            pl.BlockSpec((BR, 128), lambda i, g: (g * r + i, 0)),
            pl.BlockSpec((BR, 128), lambda i, g: (g * r + i, 0)),
            pl.BlockSpec((BR, 128), lambda i, g: (i, 0)),
            pl.BlockSpec((1, 1, 128), lambda i, g: (g, 0, 0)),
        ],
        out_specs=pl.BlockSpec((BR, 128), lambda i, g: (i, g)),
        out_shape=jax.ShapeDtypeStruct((N, 256), jnp.float32),
    )(z, y, deg16, b2g)


def kernel(x, edge_index, W1, b1, W2, b2, W3, b3):
    src = edge_index[0].astype(jnp.int32)
    dst = edge_index[1].astype(jnp.int32)

    deg16 = _deg_kernel(dst)

    y1 = _mm1(x, W1, deg16)
    z1 = _scatter4(src, dst, y1)
    y2 = _mm23(z1, y1, deg16, b1.reshape(4, 1, 128), W2, 4)
    z2 = _scatter4(src, dst, y2)
    y3 = _mm23(z2, y2, deg16, b2.reshape(4, 1, 128), W3, 2)
    z3 = _scatter2(src, dst, y3)
    return _final(z3, y3, deg16, b3.reshape(2, 1, 128))


# speedup vs baseline: 9.8865x; 1.8515x over previous
import functools

import jax
import jax.numpy as jnp
from jax import lax
from jax.experimental import pallas as pl
from jax.experimental.pallas import tpu as pltpu
from jax.experimental.pallas import tpu_sc as plsc

N = 10000
E = 160000
NT = 16
EPT = E // NT
EB = 40
RPT = 624
ZB = 16
TAIL = N - NT * RPT
BR = 1000


_SC_MESH = plsc.VectorSubcoreMesh(core_axis_name="c", subcore_axis_name="s")


def _zero_shared(s, zb_v, shared):
    def zloop(j, carry):
        pltpu.sync_copy(zb_v, shared.at[pl.ds(s * RPT + j * ZB, ZB)])
        return carry

    lax.fori_loop(0, RPT // ZB, zloop, 0)

    @pl.when(s == 0)
    def _():
        pltpu.sync_copy(zb_v.at[pl.ds(0, TAIL)], shared.at[pl.ds(NT * RPT, TAIL)])


def _write_out(s, shared, out_hbm, base):
    pltpu.sync_copy(shared.at[pl.ds(s * RPT, RPT)],
                    out_hbm.at[pl.ds(base + s * RPT, RPT)])

    @pl.when(s == 0)
    def _():
        pltpu.sync_copy(shared.at[pl.ds(NT * RPT, TAIL)],
                        out_hbm.at[pl.ds(base + NT * RPT, TAIL)])


DEB = 40


@functools.partial(
    pl.kernel,
    out_type=jax.ShapeDtypeStruct((N, 128), jnp.float32),
    mesh=_SC_MESH,
    scratch_types=[
        pltpu.VMEM((DEB,), jnp.int32),
        pltpu.VMEM((DEB, 128), jnp.float32),
        pltpu.VMEM((ZB, 128), jnp.float32),
        pltpu.VMEM_SHARED((N, 128), jnp.float32),
    ],
)
def _deg_kernel(dst_hbm, deg_hbm, dst_v, ones_v, zb_v, shared):
    c = lax.axis_index("c")
    s = lax.axis_index("s")
    one16 = jnp.full((16,), 1.0, jnp.float32)
    zero16 = jnp.zeros((16,), jnp.float32)

    def fill(i, carry):
        for j in range(8):
            ones_v[i, pl.ds(j * 16, 16)] = one16
        return carry

    lax.fori_loop(0, DEB, fill, 0)

    def fillz(i, carry):
        for j in range(8):
            zb_v[i, pl.ds(j * 16, 16)] = zero16
        return carry

    lax.fori_loop(0, ZB, fillz, 0)
    _zero_shared(s, zb_v, shared)
    plsc.subcore_barrier()

    def chunk(i, carry):
        base = s * EPT + i * DEB
        pltpu.sync_copy(dst_hbm.at[pl.ds(base, DEB)], dst_v)
        pltpu.sync_copy(ones_v, shared.at[dst_v], add=True)
        return carry

    lax.fori_loop(0, EPT // DEB, chunk, 0)
    plsc.subcore_barrier()

    @pl.when(c == 0)
    def _():
        _write_out(s, shared, deg_hbm, 0)


NCH = EPT // EB
NBUF = 5


def _make_scatter(G):
    gpc = G // 2

    @functools.partial(
        pl.kernel,
        out_type=jax.ShapeDtypeStruct((G * N, 128), jnp.float32),
        mesh=_SC_MESH,
        scratch_types=[
            pltpu.VMEM((NBUF, EB), jnp.int32),
            pltpu.VMEM((EPT,), jnp.int32),
            pltpu.VMEM((NBUF, EB, 128), jnp.float32),
            pltpu.VMEM((ZB, 128), jnp.float32),
            pltpu.VMEM_SHARED((N, 128), jnp.float32),
            pltpu.SemaphoreType.DMA((NBUF,)),
            pltpu.SemaphoreType.DMA((NBUF,)),
            pltpu.SemaphoreType.DMA((NBUF,)),
        ],
    )
    def scatter(src_hbm, dst_hbm, y_hbm, z_hbm,
                dst_r, gidx_a, rows_v, zb_v, shared, dsem, gsem, ssem):
        c = lax.axis_index("c")
        s = lax.axis_index("s")
        zero16 = jnp.zeros((16,), jnp.float32)

        def fillz(i, carry):
            for j in range(8):
                zb_v[i, pl.ds(j * 16, 16)] = zero16
            return carry

        lax.fori_loop(0, ZB, fillz, 0)
        pltpu.sync_copy(src_hbm.at[s], gidx_a)

        def fire_pair(i, b):
            pltpu.async_copy(dst_hbm.at[s, i], dst_r.at[b], dsem.at[b])
            pltpu.async_copy(y_hbm.at[gidx_a.at[pl.ds(i * EB, EB)]],
                             rows_v.at[b], gsem.at[b])

        def wait_pair(i, b):
            pltpu.make_async_copy(dst_hbm.at[s, i], dst_r.at[b],
                                  dsem.at[b]).wait()
            pltpu.make_async_copy(y_hbm.at[gidx_a.at[pl.ds(i * EB, EB)]],
                                  rows_v.at[b], gsem.at[b]).wait()

        def fire_scatter(i, b):
            pltpu.async_copy(rows_v.at[b], shared.at[dst_r.at[b]],
                             ssem.at[b], add=True)

        def wait_scatter(i, b):
            pltpu.make_async_copy(rows_v.at[b], shared.at[dst_r.at[b]],
                                  ssem.at[b]).wait()

        for gp in range(gpc):
            g = c * gpc + gp
            _zero_shared(s, zb_v, shared)
            off = c * gpc * N if gp == 0 else N

            def addoff(i, cc):
                gidx_a[pl.ds(i * 16, 16)] = gidx_a[pl.ds(i * 16, 16)] + off
                return cc

            lax.fori_loop(0, EPT // 16, addoff, 0)
            plsc.subcore_barrier()

            for b in range(NBUF):
                fire_pair(b, b)

            def outer(j, carry):
                for b in range(NBUF):
                    i = j * NBUF + b
                    wait_pair(i, b)
                    fire_scatter(i, b)
                for b in range(NBUF):
                    i = j * NBUF + b
                    wait_scatter(i, b)
                    fire_pair(i + NBUF, b)
                return carry

            nmain = NCH // NBUF - 2
            lax.fori_loop(0, nmain, outer, 0)
            for t in range(NCH - 2 * NBUF, NCH - NBUF):
                b = t % NBUF
                wait_pair(t, b)
                fire_scatter(t, b)
                wait_scatter(t, b)
                fire_pair(t + NBUF, b)
            for t in range(NCH - NBUF, NCH):
                b = t % NBUF
                wait_pair(t, b)
                fire_scatter(t, b)
                wait_scatter(t, b)

            plsc.subcore_barrier()
            _write_out(s, shared, z_hbm, g * N)
            plsc.subcore_barrier()

    return scatter


_scatter4 = _make_scatter(4)
_scatter2 = _make_scatter(2)


def _dis(deg_ref):
    return lax.rsqrt(deg_ref[:, :1] + 1.0)


def _mm1_body(x_ref, w_ref, deg_ref, y_ref):
    xw = jnp.dot(x_ref[...], w_ref[...], preferred_element_type=jnp.float32)
    y_ref[...] = xw * _dis(deg_ref)


def _mm1(x, W1, deg16):
    return pl.pallas_call(
        _mm1_body,
        grid=(N // BR, 4),
        in_specs=[
            pl.BlockSpec((BR, 256), lambda i, g: (i, 0)),
            pl.BlockSpec((256, 128), lambda i, g: (0, g)),
            pl.BlockSpec((BR, 128), lambda i, g: (i, 0)),
        ],
        out_specs=pl.BlockSpec((BR, 128), lambda i, g: (g * (N // BR) + i, 0)),
        out_shape=jax.ShapeDtypeStruct((4 * N, 128), jnp.float32),
    )(x, W1, deg16)


def _mm23_body(kgrid, z_ref, y_ref, deg_ref, b_ref, w_ref, out_ref):
    k = pl.program_id(2)
    dis = _dis(deg_ref)
    h = jnp.maximum((z_ref[...] + y_ref[...]) * dis + b_ref[0], 0.0)
    part = jnp.dot(h, w_ref[...], preferred_element_type=jnp.float32)

    @pl.when(k == 0)
    def _():
        out_ref[...] = part

    @pl.when(k > 0)
    def _():
        out_ref[...] += part

    @pl.when(k == kgrid - 1)
    def _():
        out_ref[...] *= dis


def _mm23(z, y, deg16, b4, W, gout):
    r = N // BR
    return pl.pallas_call(
        functools.partial(_mm23_body, 4),
        grid=(r, gout, 4),
        in_specs=[
            pl.BlockSpec((BR, 128), lambda i, go, k: (k * r + i, 0)),
            pl.BlockSpec((BR, 128), lambda i, go, k: (k * r + i, 0)),
            pl.BlockSpec((BR, 128), lambda i, go, k: (i, 0)),
            pl.BlockSpec((1, 1, 128), lambda i, go, k: (k, 0, 0)),
            pl.BlockSpec((128, 128), lambda i, go, k: (k, go)),
        ],
        out_specs=pl.BlockSpec((BR, 128), lambda i, go, k: (go * r + i, 0)),
        out_shape=jax.ShapeDtypeStruct((gout * N, 128), jnp.float32),
    )(z, y, deg16, b4, W)


def _final_body(z_ref, y_ref, deg_ref, b_ref, out_ref):
    out_ref[...] = jnp.maximum(
        (z_ref[...] + y_ref[...]) * _dis(deg_ref) + b_ref[0], 0.0)


def _final(z, y, deg16, b2g):
    r = N // BR
    return pl.pallas_call(
        _final_body,
        grid=(r, 2),
        in_specs=[
            pl.BlockSpec((BR, 128), lambda i, g: (g * r + i, 0)),
            pl.BlockSpec((BR, 128), lambda i, g: (g * r + i, 0)),
            pl.BlockSpec((BR, 128), lambda i, g: (i, 0)),
            pl.BlockSpec((1, 1, 128), lambda i, g: (g, 0, 0)),
        ],
        out_specs=pl.BlockSpec((BR, 128), lambda i, g: (i, g)),
        out_shape=jax.ShapeDtypeStruct((N, 256), jnp.float32),
    )(z, y, deg16, b2g)


def kernel(x, edge_index, W1, b1, W2, b2, W3, b3):
    src = edge_index[0].astype(jnp.int32).reshape(NT, EPT)
    dst = edge_index[1].astype(jnp.int32).reshape(NT, NCH, EB)

    deg16 = _deg_kernel(edge_index[1].astype(jnp.int32))

    y1 = _mm1(x, W1, deg16)
    z1 = _scatter4(src, dst, y1)
    y2 = _mm23(z1, y1, deg16, b1.reshape(4, 1, 128), W2, 4)
    z2 = _scatter4(src, dst, y2)
    y3 = _mm23(z2, y2, deg16, b2.reshape(4, 1, 128), W3, 2)
    z3 = _scatter2(src, dst, y3)
    return _final(z3, y3, deg16, b3.reshape(2, 1, 128))


# speedup vs baseline: 11.0174x; 1.1144x over previous
import functools

import jax
import jax.numpy as jnp
from jax import lax
from jax.experimental import pallas as pl
from jax.experimental.pallas import tpu as pltpu
from jax.experimental.pallas import tpu_sc as plsc

N = 10000
E = 160000
NT = 16
EPT = E // NT
EB = 40
RPT = 624
ZB = 16
TAIL = N - NT * RPT
BR = 1000


_SC_MESH = plsc.VectorSubcoreMesh(core_axis_name="c", subcore_axis_name="s")


def _zero_shared(s, zb_v, shared):
    def zloop(j, carry):
        pltpu.sync_copy(zb_v, shared.at[pl.ds(s * RPT + j * ZB, ZB)])
        return carry

    lax.fori_loop(0, RPT // ZB, zloop, 0)

    @pl.when(s == 0)
    def _():
        pltpu.sync_copy(zb_v.at[pl.ds(0, TAIL)], shared.at[pl.ds(NT * RPT, TAIL)])


def _write_out(s, shared, out_hbm, base):
    pltpu.sync_copy(shared.at[pl.ds(s * RPT, RPT)],
                    out_hbm.at[pl.ds(base + s * RPT, RPT)])

    @pl.when(s == 0)
    def _():
        pltpu.sync_copy(shared.at[pl.ds(NT * RPT, TAIL)],
                        out_hbm.at[pl.ds(base + NT * RPT, TAIL)])


DEB = 40
DNB = 5
DNC = EPT // DEB


@functools.partial(
    pl.kernel,
    out_type=jax.ShapeDtypeStruct((N, 128), jnp.float32),
    mesh=_SC_MESH,
    scratch_types=[
        pltpu.VMEM((DNB, DEB), jnp.int32),
        pltpu.VMEM((DEB, 128), jnp.float32),
        pltpu.VMEM((ZB, 128), jnp.float32),
        pltpu.VMEM_SHARED((N, 128), jnp.float32),
        pltpu.SemaphoreType.DMA((DNB,)),
        pltpu.SemaphoreType.DMA((DNB,)),
    ],
)
def _deg_kernel(dst_hbm, deg_hbm, dst_r, ones_v, zb_v, shared, dsem, ssem):
    c = lax.axis_index("c")
    s = lax.axis_index("s")
    one16 = jnp.full((16,), 1.0, jnp.float32)
    zero16 = jnp.zeros((16,), jnp.float32)

    def fill(i, carry):
        for j in range(8):
            ones_v[i, pl.ds(j * 16, 16)] = one16
        return carry

    lax.fori_loop(0, DEB, fill, 0)

    def fillz(i, carry):
        for j in range(8):
            zb_v[i, pl.ds(j * 16, 16)] = zero16
        return carry

    lax.fori_loop(0, ZB, fillz, 0)
    _zero_shared(s, zb_v, shared)
    plsc.subcore_barrier()

    def fire_dst(i, b):
        pltpu.async_copy(dst_hbm.at[s, i], dst_r.at[b], dsem.at[b])

    def wait_dst(i, b):
        pltpu.make_async_copy(dst_hbm.at[s, i], dst_r.at[b], dsem.at[b]).wait()

    def fire_add(b):
        pltpu.async_copy(ones_v, shared.at[dst_r.at[b]], ssem.at[b], add=True)

    def wait_add(b):
        pltpu.make_async_copy(ones_v, shared.at[dst_r.at[b]], ssem.at[b]).wait()

    for b in range(DNB):
        fire_dst(b, b)

    def outer(j, carry):
        for b in range(DNB):
            i = j * DNB + b
            wait_dst(i, b)
            fire_add(b)
        for b in range(DNB):
            i = j * DNB + b
            wait_add(b)
            fire_dst(i + DNB, b)
        return carry

    lax.fori_loop(0, DNC // DNB - 2, outer, 0)
    for t in range(DNC - 2 * DNB, DNC - DNB):
        b = t % DNB
        wait_dst(t, b)
        fire_add(b)
        wait_add(b)
        fire_dst(t + DNB, b)
    for t in range(DNC - DNB, DNC):
        b = t % DNB
        wait_dst(t, b)
        fire_add(b)
        wait_add(b)
    plsc.subcore_barrier()

    @pl.when(c == 0)
    def _():
        _write_out(s, shared, deg_hbm, 0)


NCH = EPT // EB
NBUF = 5


def _make_scatter(G):
    gpc = G // 2

    @functools.partial(
        pl.kernel,
        out_type=jax.ShapeDtypeStruct((G * N, 128), jnp.float32),
        mesh=_SC_MESH,
        scratch_types=[
            pltpu.VMEM((NBUF, EB), jnp.int32),
            pltpu.VMEM((EPT,), jnp.int32),
            pltpu.VMEM((NBUF, EB, 128), jnp.float32),
            pltpu.VMEM((ZB, 128), jnp.float32),
            pltpu.VMEM_SHARED((N, 128), jnp.float32),
            pltpu.SemaphoreType.DMA((NBUF,)),
            pltpu.SemaphoreType.DMA((NBUF,)),
            pltpu.SemaphoreType.DMA((NBUF,)),
        ],
    )
    def scatter(src_hbm, dst_hbm, y_hbm, z_hbm,
                dst_r, gidx_a, rows_v, zb_v, shared, dsem, gsem, ssem):
        c = lax.axis_index("c")
        s = lax.axis_index("s")
        zero16 = jnp.zeros((16,), jnp.float32)

        def fillz(i, carry):
            for j in range(8):
                zb_v[i, pl.ds(j * 16, 16)] = zero16
            return carry

        lax.fori_loop(0, ZB, fillz, 0)
        pltpu.sync_copy(src_hbm.at[s], gidx_a)

        def fire_pair(i, b):
            pltpu.async_copy(dst_hbm.at[s, i], dst_r.at[b], dsem.at[b])
            pltpu.async_copy(y_hbm.at[gidx_a.at[pl.ds(i * EB, EB)]],
                             rows_v.at[b], gsem.at[b])

        def wait_pair(i, b):
            pltpu.make_async_copy(dst_hbm.at[s, i], dst_r.at[b],
                                  dsem.at[b]).wait()
            pltpu.make_async_copy(y_hbm.at[gidx_a.at[pl.ds(i * EB, EB)]],
                                  rows_v.at[b], gsem.at[b]).wait()

        def fire_scatter(i, b):
            pltpu.async_copy(rows_v.at[b], shared.at[dst_r.at[b]],
                             ssem.at[b], add=True)

        def wait_scatter(i, b):
            pltpu.make_async_copy(rows_v.at[b], shared.at[dst_r.at[b]],
                                  ssem.at[b]).wait()

        for gp in range(gpc):
            g = c * gpc + gp
            _zero_shared(s, zb_v, shared)
            off = c * gpc * N if gp == 0 else N

            def addoff(i, cc):
                gidx_a[pl.ds(i * 16, 16)] = gidx_a[pl.ds(i * 16, 16)] + off
                return cc

            lax.fori_loop(0, EPT // 16, addoff, 0)
            plsc.subcore_barrier()

            for b in range(NBUF):
                fire_pair(b, b)

            def outer(j, carry):
                for b in range(NBUF):
                    i = j * NBUF + b
                    wait_pair(i, b)
                    fire_scatter(i, b)
                for b in range(NBUF):
                    i = j * NBUF + b
                    wait_scatter(i, b)
                    fire_pair(i + NBUF, b)
                return carry

            nmain = NCH // NBUF - 2
            lax.fori_loop(0, nmain, outer, 0)
            for t in range(NCH - 2 * NBUF, NCH - NBUF):
                b = t % NBUF
                wait_pair(t, b)
                fire_scatter(t, b)
                wait_scatter(t, b)
                fire_pair(t + NBUF, b)
            for t in range(NCH - NBUF, NCH):
                b = t % NBUF
                wait_pair(t, b)
                fire_scatter(t, b)
                wait_scatter(t, b)

            plsc.subcore_barrier()
            _write_out(s, shared, z_hbm, g * N)
            plsc.subcore_barrier()

    return scatter


_scatter4 = _make_scatter(4)
_scatter2 = _make_scatter(2)


def _dis(deg_ref):
    return lax.rsqrt(deg_ref[:, :1] + 1.0)


def _mm1_body(x_ref, w_ref, deg_ref, y_ref):
    xw = jnp.dot(x_ref[...], w_ref[...], preferred_element_type=jnp.float32)
    y_ref[...] = xw * _dis(deg_ref)


def _mm1(x, W1, deg16):
    return pl.pallas_call(
        _mm1_body,
        grid=(N // BR, 4),
        in_specs=[
            pl.BlockSpec((BR, 256), lambda i, g: (i, 0)),
            pl.BlockSpec((256, 128), lambda i, g: (0, g)),
            pl.BlockSpec((BR, 128), lambda i, g: (i, 0)),
        ],
        out_specs=pl.BlockSpec((BR, 128), lambda i, g: (g * (N // BR) + i, 0)),
        out_shape=jax.ShapeDtypeStruct((4 * N, 128), jnp.float32),
    )(x, W1, deg16)


def _mm23_body(kgrid, z_ref, y_ref, deg_ref, b_ref, w_ref, out_ref):
    k = pl.program_id(2)
    dis = _dis(deg_ref)
    h = jnp.maximum((z_ref[...] + y_ref[...]) * dis + b_ref[0], 0.0)
    part = jnp.dot(h, w_ref[...], preferred_element_type=jnp.float32)

    @pl.when(k == 0)
    def _():
        out_ref[...] = part

    @pl.when(k > 0)
    def _():
        out_ref[...] += part

    @pl.when(k == kgrid - 1)
    def _():
        out_ref[...] *= dis


def _mm23(z, y, deg16, b4, W, gout):
    r = N // BR
    return pl.pallas_call(
        functools.partial(_mm23_body, 4),
        grid=(r, gout, 4),
        in_specs=[
            pl.BlockSpec((BR, 128), lambda i, go, k: (k * r + i, 0)),
            pl.BlockSpec((BR, 128), lambda i, go, k: (k * r + i, 0)),
            pl.BlockSpec((BR, 128), lambda i, go, k: (i, 0)),
            pl.BlockSpec((1, 1, 128), lambda i, go, k: (k, 0, 0)),
            pl.BlockSpec((128, 128), lambda i, go, k: (k, go)),
        ],
        out_specs=pl.BlockSpec((BR, 128), lambda i, go, k: (go * r + i, 0)),
        out_shape=jax.ShapeDtypeStruct((gout * N, 128), jnp.float32),
    )(z, y, deg16, b4, W)


def _final_body(z_ref, y_ref, deg_ref, b_ref, out_ref):
    out_ref[...] = jnp.maximum(
        (z_ref[...] + y_ref[...]) * _dis(deg_ref) + b_ref[0], 0.0)


def _final(z, y, deg16, b2g):
    r = N // BR
    return pl.pallas_call(
        _final_body,
        grid=(r, 2),
        in_specs=[
            pl.BlockSpec((BR, 128), lambda i, g: (g * r + i, 0)),
            pl.BlockSpec((BR, 128), lambda i, g: (g * r + i, 0)),
            pl.BlockSpec((BR, 128), lambda i, g: (i, 0)),
            pl.BlockSpec((1, 1, 128), lambda i, g: (g, 0, 0)),
        ],
        out_specs=pl.BlockSpec((BR, 128), lambda i, g: (i, g)),
        out_shape=jax.ShapeDtypeStruct((N, 256), jnp.float32),
    )(z, y, deg16, b2g)


def kernel(x, edge_index, W1, b1, W2, b2, W3, b3):
    src = edge_index[0].astype(jnp.int32).reshape(NT, EPT)
    dst = edge_index[1].astype(jnp.int32).reshape(NT, NCH, EB)

    deg16 = _deg_kernel(dst)

    y1 = _mm1(x, W1, deg16)
    z1 = _scatter4(src, dst, y1)
    y2 = _mm23(z1, y1, deg16, b1.reshape(4, 1, 128), W2, 4)
    z2 = _scatter4(src, dst, y2)
    y3 = _mm23(z2, y2, deg16, b2.reshape(4, 1, 128), W3, 2)
    z3 = _scatter2(src, dst, y3)
    return _final(z3, y3, deg16, b3.reshape(2, 1, 128))


# speedup vs baseline: 12.7309x; 1.1555x over previous
import functools

import jax
import jax.numpy as jnp
from jax import lax
from jax.experimental import pallas as pl
from jax.experimental.pallas import tpu as pltpu
from jax.experimental.pallas import tpu_sc as plsc

N = 10000
E = 160000
NT = 16
EPT = E // NT
EB = 40
RPT = 624
ZB = 16
TAIL = N - NT * RPT
BR = 1000


_SC_MESH = plsc.VectorSubcoreMesh(core_axis_name="c", subcore_axis_name="s")


def _zero_shared(s, zb_v, shared):
    def zloop(j, carry):
        pltpu.sync_copy(zb_v, shared.at[pl.ds(s * RPT + j * ZB, ZB)])
        return carry

    lax.fori_loop(0, RPT // ZB, zloop, 0)

    @pl.when(s == 0)
    def _():
        pltpu.sync_copy(zb_v.at[pl.ds(0, TAIL)], shared.at[pl.ds(NT * RPT, TAIL)])


def _write_out(s, shared, out_hbm, col):
    cs = pl.ds(col, 128)
    pltpu.sync_copy(shared.at[pl.ds(s * RPT, RPT)],
                    out_hbm.at[pl.ds(s * RPT, RPT), cs])

    @pl.when(s == 0)
    def _():
        pltpu.sync_copy(shared.at[pl.ds(NT * RPT, TAIL)],
                        out_hbm.at[pl.ds(NT * RPT, TAIL), cs])


DEB = 40
DNB = 5
DNC = EPT // DEB


@functools.partial(
    pl.kernel,
    out_type=jax.ShapeDtypeStruct((N, 128), jnp.float32),
    mesh=_SC_MESH,
    scratch_types=[
        pltpu.VMEM((DNB, DEB), jnp.int32),
        pltpu.VMEM((DEB, 128), jnp.float32),
        pltpu.VMEM((ZB, 128), jnp.float32),
        pltpu.VMEM_SHARED((N, 128), jnp.float32),
        pltpu.SemaphoreType.DMA((DNB,)),
        pltpu.SemaphoreType.DMA((DNB,)),
    ],
)
def _deg_kernel(dst_hbm, deg_hbm, dst_r, ones_v, zb_v, shared, dsem, ssem):
    c = lax.axis_index("c")
    s = lax.axis_index("s")
    one16 = jnp.full((16,), 1.0, jnp.float32)
    zero16 = jnp.zeros((16,), jnp.float32)

    def fill(i, carry):
        for j in range(8):
            ones_v[i, pl.ds(j * 16, 16)] = one16
        return carry

    lax.fori_loop(0, DEB, fill, 0)

    def fillz(i, carry):
        for j in range(8):
            zb_v[i, pl.ds(j * 16, 16)] = zero16
        return carry

    lax.fori_loop(0, ZB, fillz, 0)
    _zero_shared(s, zb_v, shared)
    plsc.subcore_barrier()

    def fire_dst(i, b):
        pltpu.async_copy(dst_hbm.at[s, i], dst_r.at[b], dsem.at[b])

    def wait_dst(i, b):
        pltpu.make_async_copy(dst_hbm.at[s, i], dst_r.at[b], dsem.at[b]).wait()

    def fire_add(b):
        pltpu.async_copy(ones_v, shared.at[dst_r.at[b]], ssem.at[b], add=True)

    def wait_add(b):
        pltpu.make_async_copy(ones_v, shared.at[dst_r.at[b]], ssem.at[b]).wait()

    for b in range(DNB):
        fire_dst(b, b)

    def outer(j, carry):
        for b in range(DNB):
            i = j * DNB + b
            wait_dst(i, b)
            fire_add(b)
        for b in range(DNB):
            i = j * DNB + b
            wait_add(b)
            fire_dst(i + DNB, b)
        return carry

    lax.fori_loop(0, DNC // DNB - 2, outer, 0)
    for t in range(DNC - 2 * DNB, DNC - DNB):
        b = t % DNB
        wait_dst(t, b)
        fire_add(b)
        wait_add(b)
        fire_dst(t + DNB, b)
    for t in range(DNC - DNB, DNC):
        b = t % DNB
        wait_dst(t, b)
        fire_add(b)
        wait_add(b)
    plsc.subcore_barrier()

    @pl.when(c == 0)
    def _():
        _write_out(s, shared, deg_hbm, 0)


NCH = EPT // EB
NBUF = 5


def _make_scatter(G):
    gpc = G // 2

    @functools.partial(
        pl.kernel,
        out_type=jax.ShapeDtypeStruct((N, G * 128), jnp.float32),
        mesh=_SC_MESH,
        scratch_types=[
            pltpu.VMEM((NBUF, EB), jnp.int32),
            pltpu.VMEM((EPT,), jnp.int32),
            pltpu.VMEM((NBUF, EB, 128), jnp.float32),
            pltpu.VMEM((ZB, 128), jnp.float32),
            pltpu.VMEM_SHARED((N, 128), jnp.float32),
            pltpu.SemaphoreType.DMA((NBUF,)),
            pltpu.SemaphoreType.DMA((NBUF,)),
            pltpu.SemaphoreType.DMA((NBUF,)),
        ],
    )
    def scatter(src_hbm, dst_hbm, y_hbm, z_hbm,
                dst_r, gidx_a, rows_v, zb_v, shared, dsem, gsem, ssem):
        c = lax.axis_index("c")
        s = lax.axis_index("s")
        zero16 = jnp.zeros((16,), jnp.float32)

        def fillz(i, carry):
            for j in range(8):
                zb_v[i, pl.ds(j * 16, 16)] = zero16
            return carry

        lax.fori_loop(0, ZB, fillz, 0)
        pltpu.sync_copy(src_hbm.at[s], gidx_a)

        def fire_pair(i, b):
            pltpu.async_copy(dst_hbm.at[s, i], dst_r.at[b], dsem.at[b])
            pltpu.async_copy(y_hbm.at[gidx_a.at[pl.ds(i * EB, EB)]],
                             rows_v.at[b], gsem.at[b])

        def wait_pair(i, b):
            pltpu.make_async_copy(dst_hbm.at[s, i], dst_r.at[b],
                                  dsem.at[b]).wait()
            pltpu.make_async_copy(y_hbm.at[gidx_a.at[pl.ds(i * EB, EB)]],
                                  rows_v.at[b], gsem.at[b]).wait()

        def fire_scatter(i, b):
            pltpu.async_copy(rows_v.at[b], shared.at[dst_r.at[b]],
                             ssem.at[b], add=True)

        def wait_scatter(i, b):
            pltpu.make_async_copy(rows_v.at[b], shared.at[dst_r.at[b]],
                                  ssem.at[b]).wait()

        for gp in range(gpc):
            g = c * gpc + gp
            _zero_shared(s, zb_v, shared)
            off = c * gpc * N if gp == 0 else N

            def addoff(i, cc):
                gidx_a[pl.ds(i * 16, 16)] = gidx_a[pl.ds(i * 16, 16)] + off
                return cc

            lax.fori_loop(0, EPT // 16, addoff, 0)
            plsc.subcore_barrier()

            for b in range(NBUF):
                fire_pair(b, b)

            def outer(j, carry):
                for b in range(NBUF):
                    i = j * NBUF + b
                    wait_pair(i, b)
                    fire_scatter(i, b)
                for b in range(NBUF):
                    i = j * NBUF + b
                    wait_scatter(i, b)
                    fire_pair(i + NBUF, b)
                return carry

            nmain = NCH // NBUF - 2
            lax.fori_loop(0, nmain, outer, 0)
            for t in range(NCH - 2 * NBUF, NCH - NBUF):
                b = t % NBUF
                wait_pair(t, b)
                fire_scatter(t, b)
                wait_scatter(t, b)
                fire_pair(t + NBUF, b)
            for t in range(NCH - NBUF, NCH):
                b = t % NBUF
                wait_pair(t, b)
                fire_scatter(t, b)
                wait_scatter(t, b)

            plsc.subcore_barrier()
            _write_out(s, shared, z_hbm, pl.multiple_of(g * 128, 128))
            plsc.subcore_barrier()

    return scatter


_scatter4 = _make_scatter(4)
_scatter2 = _make_scatter(2)


def _dis(deg_ref):
    return lax.rsqrt(deg_ref[:, :1] + 1.0)


def _mm1_body(x_ref, w_ref, deg_ref, yf_ref, yn_ref):
    xw = jnp.dot(x_ref[...], w_ref[...], preferred_element_type=jnp.float32)
    yv = xw * _dis(deg_ref)
    yf_ref[...] = yv
    yn_ref[...] = yv


def _mm1(x, W1, deg16):
    r = N // BR
    return pl.pallas_call(
        _mm1_body,
        grid=(r, 4),
        in_specs=[
            pl.BlockSpec((BR, 256), lambda i, g: (i, 0)),
            pl.BlockSpec((256, 128), lambda i, g: (0, g)),
            pl.BlockSpec((BR, 128), lambda i, g: (i, 0)),
        ],
        out_specs=[
            pl.BlockSpec((BR, 128), lambda i, g: (g * (N // BR) + i, 0)),
            pl.BlockSpec((BR, 128), lambda i, g: (i, g)),
        ],
        out_shape=[
            jax.ShapeDtypeStruct((4 * N, 128), jnp.float32),
            jax.ShapeDtypeStruct((N, 512), jnp.float32),
        ],
    )(x, W1, deg16)


def _mm23_body(z_ref, y_ref, deg_ref, b_ref, w_ref, yf_ref, yn_ref, hc_ref):
    go = pl.program_id(1)
    dis = _dis(deg_ref)

    @pl.when(go == 0)
    def _():
        hc_ref[...] = jnp.maximum(
            (z_ref[...] + y_ref[...]) * dis + b_ref[0], 0.0)

    yv = jnp.dot(hc_ref[...], w_ref[...],
                 preferred_element_type=jnp.float32) * dis
    yf_ref[...] = yv
    yn_ref[...] = yv


def _mm23(z, y, deg16, b2d, W, gout):
    r = N // BR
    din = W.shape[0]
    return pl.pallas_call(
        _mm23_body,
        grid=(r, gout),
        in_specs=[
            pl.BlockSpec((BR, din), lambda i, go: (i, 0)),
            pl.BlockSpec((BR, din), lambda i, go: (i, 0)),
            pl.BlockSpec((BR, 128), lambda i, go: (i, 0)),
            pl.BlockSpec((1, din), lambda i, go: (0, 0)),
            pl.BlockSpec((din, 128), lambda i, go: (0, go)),
        ],
        out_specs=[
            pl.BlockSpec((BR, 128), lambda i, go: (go * (N // BR) + i, 0)),
            pl.BlockSpec((BR, 128), lambda i, go: (i, go)),
        ],
        out_shape=[
            jax.ShapeDtypeStruct((gout * N, 128), jnp.float32),
            jax.ShapeDtypeStruct((N, gout * 128), jnp.float32),
        ],
        scratch_shapes=[pltpu.VMEM((BR, din), jnp.float32)],
    )(z, y, deg16, b2d, W)


def _final_body(z_ref, y_ref, deg_ref, b_ref, out_ref):
    out_ref[...] = jnp.maximum(
        (z_ref[...] + y_ref[...]) * _dis(deg_ref) + b_ref[0], 0.0)


def _final(z, y, deg16, b2d):
    r = N // BR
    return pl.pallas_call(
        _final_body,
        grid=(r,),
        in_specs=[
            pl.BlockSpec((BR, 256), lambda i: (i, 0)),
            pl.BlockSpec((BR, 256), lambda i: (i, 0)),
            pl.BlockSpec((BR, 128), lambda i: (i, 0)),
            pl.BlockSpec((1, 256), lambda i: (0, 0)),
        ],
        out_specs=pl.BlockSpec((BR, 256), lambda i: (i, 0)),
        out_shape=jax.ShapeDtypeStruct((N, 256), jnp.float32),
    )(z, y, deg16, b2d)


def kernel(x, edge_index, W1, b1, W2, b2, W3, b3):
    src = edge_index[0].astype(jnp.int32).reshape(NT, EPT)
    dst = edge_index[1].astype(jnp.int32).reshape(NT, NCH, EB)

    deg16 = _deg_kernel(dst)

    y1f, y1n = _mm1(x, W1, deg16)
    z1 = _scatter4(src, dst, y1f)
    y2f, y2n = _mm23(z1, y1n, deg16, b1.reshape(1, 512), W2, 4)
    z2 = _scatter4(src, dst, y2f)
    y3f, y3n = _mm23(z2, y2n, deg16, b2.reshape(1, 512), W3, 2)
    z3 = _scatter2(src, dst, y3f)
    return _final(z3, y3n, deg16, b3.reshape(1, 256))


# speedup vs baseline: 12.8514x; 1.0095x over previous
import functools

import jax
import jax.numpy as jnp
from jax import lax
from jax.experimental import pallas as pl
from jax.experimental.pallas import tpu as pltpu
from jax.experimental.pallas import tpu_sc as plsc

N = 10000
E = 160000
NT = 16
EPT = E // NT
EB = 40
RPT = 624
ZB = 48
TAIL = N - NT * RPT
BR = 1000


_SC_MESH = plsc.VectorSubcoreMesh(core_axis_name="c", subcore_axis_name="s")


def _zero_shared(s, zb_v, shared):
    def zloop(j, carry):
        pltpu.sync_copy(zb_v, shared.at[pl.ds(s * RPT + j * ZB, ZB)])
        return carry

    lax.fori_loop(0, RPT // ZB, zloop, 0)

    @pl.when(s == 0)
    def _():
        pltpu.sync_copy(zb_v.at[pl.ds(0, TAIL)], shared.at[pl.ds(NT * RPT, TAIL)])


def _write_out(s, shared, out_hbm, col):
    cs = pl.ds(col, 128)
    pltpu.sync_copy(shared.at[pl.ds(s * RPT, RPT)],
                    out_hbm.at[pl.ds(s * RPT, RPT), cs])

    @pl.when(s == 0)
    def _():
        pltpu.sync_copy(shared.at[pl.ds(NT * RPT, TAIL)],
                        out_hbm.at[pl.ds(NT * RPT, TAIL), cs])


DEB = 40
DNB = 5
DNC = EPT // DEB


@functools.partial(
    pl.kernel,
    out_type=jax.ShapeDtypeStruct((N, 128), jnp.float32),
    mesh=_SC_MESH,
    scratch_types=[
        pltpu.VMEM((DNB, DEB), jnp.int32),
        pltpu.VMEM((DEB, 128), jnp.float32),
        pltpu.VMEM((ZB, 128), jnp.float32),
        pltpu.VMEM_SHARED((N, 128), jnp.float32),
        pltpu.SemaphoreType.DMA((DNB,)),
        pltpu.SemaphoreType.DMA((DNB,)),
    ],
)
def _deg_kernel(dst_hbm, deg_hbm, dst_r, ones_v, zb_v, shared, dsem, ssem):
    c = lax.axis_index("c")
    s = lax.axis_index("s")
    one16 = jnp.full((16,), 1.0, jnp.float32)
    zero16 = jnp.zeros((16,), jnp.float32)

    def fill(i, carry):
        for j in range(8):
            ones_v[i, pl.ds(j * 16, 16)] = one16
        return carry

    lax.fori_loop(0, DEB, fill, 0)

    def fillz(i, carry):
        for j in range(8):
            zb_v[i, pl.ds(j * 16, 16)] = zero16
        return carry

    lax.fori_loop(0, ZB, fillz, 0)
    _zero_shared(s, zb_v, shared)
    plsc.subcore_barrier()

    def fire_dst(i, b):
        pltpu.async_copy(dst_hbm.at[s, i], dst_r.at[b], dsem.at[b])

    def wait_dst(i, b):
        pltpu.make_async_copy(dst_hbm.at[s, i], dst_r.at[b], dsem.at[b]).wait()

    def fire_add(b):
        pltpu.async_copy(ones_v, shared.at[dst_r.at[b]], ssem.at[b], add=True)

    def wait_add(b):
        pltpu.make_async_copy(ones_v, shared.at[dst_r.at[b]], ssem.at[b]).wait()

    for b in range(DNB):
        fire_dst(b, b)

    def outer(j, carry):
        for b in range(DNB):
            i = j * DNB + b
            wait_dst(i, b)
            fire_add(b)
        for b in range(DNB):
            i = j * DNB + b
            wait_add(b)
            fire_dst(i + DNB, b)
        return carry

    lax.fori_loop(0, DNC // DNB - 2, outer, 0)
    for t in range(DNC - 2 * DNB, DNC - DNB):
        b = t % DNB
        wait_dst(t, b)
        fire_add(b)
        wait_add(b)
        fire_dst(t + DNB, b)
    for t in range(DNC - DNB, DNC):
        b = t % DNB
        wait_dst(t, b)
        fire_add(b)
        wait_add(b)
    plsc.subcore_barrier()

    @pl.when(c == 0)
    def _():
        _write_out(s, shared, deg_hbm, 0)


NCH = EPT // EB
NBUF = 5


def _make_scatter(G):
    gpc = G // 2

    @functools.partial(
        pl.kernel,
        out_type=jax.ShapeDtypeStruct((N, G * 128), jnp.float32),
        mesh=_SC_MESH,
        scratch_types=[
            pltpu.VMEM((NBUF, EB), jnp.int32),
            pltpu.VMEM((EPT,), jnp.int32),
            pltpu.VMEM((NBUF, EB, 128), jnp.float32),
            pltpu.VMEM((ZB, 128), jnp.float32),
            pltpu.VMEM_SHARED((N, 128), jnp.float32),
            pltpu.SemaphoreType.DMA((NBUF,)),
            pltpu.SemaphoreType.DMA((NBUF,)),
            pltpu.SemaphoreType.DMA((NBUF,)),
        ],
    )
    def scatter(src_hbm, dst_hbm, y_hbm, z_hbm,
                dst_r, gidx_a, rows_v, zb_v, shared, dsem, gsem, ssem):
        c = lax.axis_index("c")
        s = lax.axis_index("s")
        zero16 = jnp.zeros((16,), jnp.float32)

        def fillz(i, carry):
            for j in range(8):
                zb_v[i, pl.ds(j * 16, 16)] = zero16
            return carry

        lax.fori_loop(0, ZB, fillz, 0)
        pltpu.sync_copy(src_hbm.at[s], gidx_a)

        def fire_pair(i, b):
            pltpu.async_copy(dst_hbm.at[s, i], dst_r.at[b], dsem.at[b])
            pltpu.async_copy(y_hbm.at[gidx_a.at[pl.ds(i * EB, EB)]],
                             rows_v.at[b], gsem.at[b])

        def wait_pair(i, b):
            pltpu.make_async_copy(dst_hbm.at[s, i], dst_r.at[b],
                                  dsem.at[b]).wait()
            pltpu.make_async_copy(y_hbm.at[gidx_a.at[pl.ds(i * EB, EB)]],
                                  rows_v.at[b], gsem.at[b]).wait()

        def fire_scatter(i, b):
            pltpu.async_copy(rows_v.at[b], shared.at[dst_r.at[b]],
                             ssem.at[b], add=True)

        def wait_scatter(i, b):
            pltpu.make_async_copy(rows_v.at[b], shared.at[dst_r.at[b]],
                                  ssem.at[b]).wait()

        for gp in range(gpc):
            g = c * gpc + gp
            _zero_shared(s, zb_v, shared)
            off = c * gpc * N if gp == 0 else N

            def addoff(i, cc):
                gidx_a[pl.ds(i * 16, 16)] = gidx_a[pl.ds(i * 16, 16)] + off
                return cc

            lax.fori_loop(0, EPT // 16, addoff, 0)
            plsc.subcore_barrier()

            for b in range(NBUF):
                fire_pair(b, b)

            def outer(j, carry):
                for b in range(NBUF):
                    i = j * NBUF + b
                    wait_pair(i, b)
                    fire_scatter(i, b)
                for b in range(NBUF):
                    i = j * NBUF + b
                    wait_scatter(i, b)
                    fire_pair(i + NBUF, b)
                return carry

            nmain = NCH // NBUF - 2
            lax.fori_loop(0, nmain, outer, 0)
            for t in range(NCH - 2 * NBUF, NCH - NBUF):
                b = t % NBUF
                wait_pair(t, b)
                fire_scatter(t, b)
                wait_scatter(t, b)
                fire_pair(t + NBUF, b)
            for t in range(NCH - NBUF, NCH):
                b = t % NBUF
                wait_pair(t, b)
                fire_scatter(t, b)
                wait_scatter(t, b)

            plsc.subcore_barrier()
            _write_out(s, shared, z_hbm, pl.multiple_of(g * 128, 128))
            plsc.subcore_barrier()

    return scatter


_scatter4 = _make_scatter(4)
_scatter2 = _make_scatter(2)


def _dis(deg_ref):
    return lax.rsqrt(deg_ref[:, :1] + 1.0)


def _mm1_body(x_ref, w_ref, deg_ref, yf_ref, yn_ref):
    xw = jnp.dot(x_ref[...], w_ref[...], preferred_element_type=jnp.float32)
    yv = xw * _dis(deg_ref)
    yf_ref[...] = yv
    yn_ref[...] = yv


def _mm1(x, W1, deg16):
    r = N // BR
    return pl.pallas_call(
        _mm1_body,
        grid=(r, 4),
        in_specs=[
            pl.BlockSpec((BR, 256), lambda i, g: (i, 0)),
            pl.BlockSpec((256, 128), lambda i, g: (0, g)),
            pl.BlockSpec((BR, 128), lambda i, g: (i, 0)),
        ],
        out_specs=[
            pl.BlockSpec((BR, 128), lambda i, g: (g * (N // BR) + i, 0)),
            pl.BlockSpec((BR, 128), lambda i, g: (i, g)),
        ],
        out_shape=[
            jax.ShapeDtypeStruct((4 * N, 128), jnp.float32),
            jax.ShapeDtypeStruct((N, 512), jnp.float32),
        ],
    )(x, W1, deg16)


def _mm23_body(z_ref, y_ref, deg_ref, b_ref, w_ref, yf_ref, yn_ref, hc_ref):
    go = pl.program_id(1)
    dis = _dis(deg_ref)

    @pl.when(go == 0)
    def _():
        hc_ref[...] = jnp.maximum(
            (z_ref[...] + y_ref[...]) * dis + b_ref[0], 0.0)

    yv = jnp.dot(hc_ref[...], w_ref[...],
                 preferred_element_type=jnp.float32) * dis
    yf_ref[...] = yv
    yn_ref[...] = yv


def _mm23(z, y, deg16, b2d, W, gout):
    r = N // BR
    din = W.shape[0]
    return pl.pallas_call(
        _mm23_body,
        grid=(r, gout),
        in_specs=[
            pl.BlockSpec((BR, din), lambda i, go: (i, 0)),
            pl.BlockSpec((BR, din), lambda i, go: (i, 0)),
            pl.BlockSpec((BR, 128), lambda i, go: (i, 0)),
            pl.BlockSpec((1, din), lambda i, go: (0, 0)),
            pl.BlockSpec((din, 128), lambda i, go: (0, go)),
        ],
        out_specs=[
            pl.BlockSpec((BR, 128), lambda i, go: (go * (N // BR) + i, 0)),
            pl.BlockSpec((BR, 128), lambda i, go: (i, go)),
        ],
        out_shape=[
            jax.ShapeDtypeStruct((gout * N, 128), jnp.float32),
            jax.ShapeDtypeStruct((N, gout * 128), jnp.float32),
        ],
        scratch_shapes=[pltpu.VMEM((BR, din), jnp.float32)],
    )(z, y, deg16, b2d, W)


def _final_body(z_ref, y_ref, deg_ref, b_ref, out_ref):
    out_ref[...] = jnp.maximum(
        (z_ref[...] + y_ref[...]) * _dis(deg_ref) + b_ref[0], 0.0)


def _final(z, y, deg16, b2d):
    r = N // BR
    return pl.pallas_call(
        _final_body,
        grid=(r,),
        in_specs=[
            pl.BlockSpec((BR, 256), lambda i: (i, 0)),
            pl.BlockSpec((BR, 256), lambda i: (i, 0)),
            pl.BlockSpec((BR, 128), lambda i: (i, 0)),
            pl.BlockSpec((1, 256), lambda i: (0, 0)),
        ],
        out_specs=pl.BlockSpec((BR, 256), lambda i: (i, 0)),
        out_shape=jax.ShapeDtypeStruct((N, 256), jnp.float32),
    )(z, y, deg16, b2d)


def kernel(x, edge_index, W1, b1, W2, b2, W3, b3):
    src = edge_index[0].astype(jnp.int32).reshape(NT, EPT)
    dst = edge_index[1].astype(jnp.int32).reshape(NT, NCH, EB)

    dstd = edge_index[1].astype(jnp.int32).reshape(NT, DNC, DEB)
    deg16 = _deg_kernel(dstd)

    y1f, y1n = _mm1(x, W1, deg16)
    z1 = _scatter4(src, dst, y1f)
    y2f, y2n = _mm23(z1, y1n, deg16, b1.reshape(1, 512), W2, 4)
    z2 = _scatter4(src, dst, y2f)
    y3f, y3n = _mm23(z2, y2n, deg16, b2.reshape(1, 512), W3, 2)
    z3 = _scatter2(src, dst, y3f)
    return _final(z3, y3n, deg16, b3.reshape(1, 256))
